# baseline (speedup 1.0000x reference)
import functools

import jax
import jax.numpy as jnp
from jax import lax
from jax.experimental import pallas as pl
from jax.experimental.pallas import tpu as pltpu
from jax.experimental.pallas import tpu_sc as plsc

N = 10000
E = 320000
D_IN = 128
H = 16
C = 40

NP = 10240
F2 = 48
E2 = 327680

NC = 2
NS = 16
NW = NC * NS
EPW = E2 // NW
K = 128
CH = EPW // K
CHD = (E2 // NS) // K
CHQ = CH // 4
RP = NP // NS
DEG_Q = 16

_mesh = plsc.VectorSubcoreMesh(core_axis_name="c", subcore_axis_name="s")
_sc_params = pltpu.CompilerParams(use_tc_tiling_on_sc=False,
                                  needs_layout_passes=False)


def _agg_pipeline(tab, idxr_v, idxc_v, buf, acc_sp, semg, sems):
  def g(j, q):
    pltpu.async_copy(tab.at[idxr_v.at[j]], buf.at[q], semg[q])

  def sct(j, p):
    pltpu.async_copy(buf.at[p], acc_sp.at[idxc_v.at[j]], sems[p], add=True)

  def wait_g(q):
    pltpu.make_async_copy(tab.at[pl.ds(0, K)], buf.at[q], semg[q]).wait()

  def wait_s(p):
    pltpu.make_async_copy(buf.at[p], acc_sp.at[pl.ds(0, K)], sems[p]).wait()

  g(0, 0)
  g(1, 1)
  wait_g(0); sct(0, 0); g(2, 2)
  wait_g(1); sct(1, 1); g(3, 3)
  wait_g(2); sct(2, 2); wait_s(0); g(4, 0)
  wait_g(3); sct(3, 3); wait_s(1); g(5, 1)

  def body(t, _):
    for p in range(4):
      j = 4 * t + p
      q = (p + 2) % 4
      wait_g(p)
      sct(j, p)
      wait_s(q)
      g(j + 2, q)
    return 0

  lax.fori_loop(1, CHQ - 1, body, 0)

  for p in range(4):
    j = 4 * (CHQ - 1) + p
    q = (p + 2) % 4
    wait_g(p)
    sct(j, p)
    wait_s(q)
    if j + 2 < CH:
      g(j + 2, q)
  wait_s(2)
  wait_s(3)


def _sc_fused_layer1(row2d, col2d, h1, zeros16):
  @functools.partial(
      pl.kernel,
      out_type=(jax.ShapeDtypeStruct((NC, NP, H), jnp.float32),
                jax.ShapeDtypeStruct((NC, NP, H), jnp.float32)),
      mesh=_mesh,
      compiler_params=_sc_params,
      scratch_types=[
          pltpu.VMEM((CHD, K), jnp.int32),
          pltpu.VMEM((CH, K), jnp.int32),
          pltpu.VMEM((CH, K), jnp.int32),
          pltpu.VMEM((4, K, H), jnp.float32),
          pltpu.VMEM((RP, H), jnp.float32),
          pltpu.VMEM((RP, H), jnp.float32),
          pltpu.VMEM((RP, H), jnp.float32),
          pltpu.VMEM((RP, H), jnp.float32),
          pltpu.VMEM((RP,), jnp.float32),
          pltpu.VMEM((RP, H), jnp.float32),
          pltpu.VMEM((K, H), jnp.float32),
          pltpu.VMEM_SHARED((NP, H), jnp.float32),
          pltpu.VMEM_SHARED((NP, H), jnp.float32),
      ] + [pltpu.SemaphoreType.DMA] * 13,
  )
  def k(row_hbm, col_hbm, h1_hbm, z_hbm, agg_out, dis_out,
        idxd, idxr_v, idxc_v, buf, degst, h1v, h1pv, disrow, disf,
        zerov, onesv, acc_sp, h1p_sp, *sems13):
    si0, si1, si2, si3, semd = sems13[0:5]
    semg = sems13[5:9]
    sems = sems13[9:13]
    c = lax.axis_index("c")
    s = lax.axis_index("s")
    w = s * NC + c

    ld0 = pltpu.async_copy(col_hbm.at[pl.ds(s * CHD, CHD)], idxd, si0)
    ld1 = pltpu.async_copy(row_hbm.at[pl.ds(w * CH, CH)], idxr_v, si1)
    ld2 = pltpu.async_copy(col_hbm.at[pl.ds(w * CH, CH)], idxc_v, si2)
    ld3 = pltpu.async_copy(h1_hbm.at[pl.ds(s * RP, RP)], h1v, si3)

    def fill_ones(i, _):
      onesv[i, :] = jnp.ones((H,), jnp.float32)
      return 0

    lax.fori_loop(0, K, fill_ones, 0)
    pltpu.sync_copy(z_hbm.at[pl.ds(s * RP, RP)], zerov)
    pltpu.sync_copy(zerov, acc_sp.at[pl.ds(s * RP, RP)])
    ld0.wait()
    plsc.subcore_barrier()

    def wait_one_deg():
      pltpu.make_async_copy(onesv, acc_sp.at[pl.ds(0, K)], semd).wait()

    def dbody(j, _):
      pltpu.async_copy(onesv, acc_sp.at[idxd.at[j]], semd, add=True)

      @pl.when(j >= DEG_Q)
      def _():
        wait_one_deg()

      return 0

    lax.fori_loop(0, CHD, dbody, 0)

    def drain(j, _):
      wait_one_deg()
      return 0

    lax.fori_loop(0, DEG_Q, drain, 0)
    plsc.subcore_barrier()

    pltpu.sync_copy(acc_sp.at[pl.ds(s * RP, RP)], degst)
    iota = lax.iota(jnp.int32, 16)
    zi = jnp.zeros((16,), jnp.int32)

    def disbody(i, _):
      v = plsc.load_gather(degst, [i * 16 + iota, zi])
      bits = lax.bitcast_convert_type(v, jnp.int32)
      y = lax.bitcast_convert_type(jnp.int32(0x5F3759DF) - (bits >> 1),
                                   jnp.float32)
      for _ in range(4):
        y = y * (1.5 - 0.5 * v * y * y)
      y = jnp.where(v > 0, y, 0.0)
      disf[pl.ds(i * 16, 16)] = y
      return 0

    lax.fori_loop(0, RP // 16, disbody, 0)

    ld3.wait()

    def scale(r, _):
      db = plsc.load_gather(disf, [jnp.full((16,), r, jnp.int32)])
      h1pv[r, :] = h1v[r, :] * db
      disrow[r, :] = db
      return 0

    lax.fori_loop(0, RP, scale, 0)
    pltpu.sync_copy(h1pv, h1p_sp.at[pl.ds(s * RP, RP)])
    pltpu.sync_copy(disrow, dis_out.at[c, pl.ds(s * RP, RP)])
    ld1.wait()
    ld2.wait()
    plsc.subcore_barrier()
    pltpu.sync_copy(zerov, acc_sp.at[pl.ds(s * RP, RP)])
    plsc.subcore_barrier()

    _agg_pipeline(h1p_sp, idxr_v, idxc_v, buf, acc_sp, semg, sems)
    plsc.subcore_barrier()
    pltpu.sync_copy(acc_sp.at[pl.ds(s * RP, RP)],
                    agg_out.at[c, pl.ds(s * RP, RP)])

  return k(row2d, col2d, h1, zeros16)


def _sc_agg(row2d, col2d, table, zeros, f):
  @functools.partial(
      pl.kernel,
      out_type=jax.ShapeDtypeStruct((NC, NP, f), jnp.float32),
      mesh=_mesh,
      compiler_params=_sc_params,
      scratch_types=[
          pltpu.VMEM((CH, K), jnp.int32),
          pltpu.VMEM((CH, K), jnp.int32),
          pltpu.VMEM((4, K, f), jnp.float32),
          pltpu.VMEM((RP, f), jnp.float32),
          pltpu.VMEM_SHARED((NP, f), jnp.float32),
      ] + [pltpu.SemaphoreType.DMA] * 8,
  )
  def k(row_hbm, col_hbm, tab_hbm, z_hbm, out_hbm,
        idxr_v, idxc_v, buf, zero_v, acc_sp, *sems8):
    semg = sems8[0:4]
    sems = sems8[4:8]
    c = lax.axis_index("c")
    s = lax.axis_index("s")
    w = s * NC + c

    ldr = pltpu.async_copy(row_hbm.at[pl.ds(w * CH, CH)], idxr_v, semg[0])
    ldc = pltpu.async_copy(col_hbm.at[pl.ds(w * CH, CH)], idxc_v, semg[1])
    pltpu.sync_copy(z_hbm.at[pl.ds(s * RP, RP)], zero_v)
    pltpu.sync_copy(zero_v, acc_sp.at[pl.ds(s * RP, RP)])
    ldr.wait()
    ldc.wait()
    plsc.subcore_barrier()

    _agg_pipeline(tab_hbm, idxr_v, idxc_v, buf, acc_sp, semg, sems)
    plsc.subcore_barrier()
    pltpu.sync_copy(acc_sp.at[pl.ds(s * RP, RP)],
                    out_hbm.at[c, pl.ds(s * RP, RP)])

  return k(row2d, col2d, table, zeros)


def _tc1(x_ref, w1_ref, w01_ref, w02_ref, h1_ref, sw1_ref, sw2_ref):
  h1_ref[pl.ds(0, N), :] = jnp.dot(x_ref[...], w1_ref[...],
                                   preferred_element_type=jnp.float32)
  h1_ref[pl.ds(N, NP - N), :] = jnp.zeros((NP - N, H), jnp.float32)
  sw1_ref[...] = jax.nn.sigmoid(w01_ref[...])
  sw2_ref[...] = jax.nn.sigmoid(w02_ref[...])


def _tc3(disb_ref, aggp_ref, h1_ref, sw_ref, b1_ref, w2_ref,
         h2_ref, h2p_ref):
  dis16 = disb_ref[0]
  agg = aggp_ref[0] + aggp_ref[1]
  z = dis16 * agg + sw_ref[...] * h1_ref[...] + b1_ref[...]
  z = jnp.maximum(z, 0.0)
  h2 = jnp.dot(z, w2_ref[...], preferred_element_type=jnp.float32)
  h2_ref[...] = h2
  h2p_ref[...] = h2 * dis16[:, 0:1]


def _tc4(disb_ref, aggp_ref, h2_ref, sw_ref, b2_ref, o_ref):
  dis1 = disb_ref[0][:, 0:1]
  agg = aggp_ref[0] + aggp_ref[1]
  out = dis1 * agg + sw_ref[...] * h2_ref[...] + b2_ref[...]
  o_ref[...] = out[:N, :C]


def kernel(x, edge_index, w0_1, W1, b1, w0_2, W2, b2):
  ei = edge_index.astype(jnp.int32)
  eip = jnp.pad(ei, ((0, 0), (0, E2 - E)), constant_values=NP - 1)
  row2d = eip[0].reshape(NW * CH, K)
  col2d = eip[1].reshape(NW * CH, K)

  w0_1q = jnp.pad(w0_1, (0, NP - N)).reshape(NP // 128, 128)
  w0_2q = jnp.pad(w0_2, (0, NP - N)).reshape(NP // 128, 128)
  W2p = jnp.pad(W2, ((0, 0), (0, F2 - C)))
  b1r = b1.reshape(1, H)
  b2r = jnp.pad(b2, (0, F2 - C)).reshape(1, F2)
  zeros48 = jnp.zeros((NP, F2), jnp.float32)
  zeros16 = jnp.zeros((NP, H), jnp.float32)

  f32 = jnp.float32
  h1, sw1q, sw2q = pl.pallas_call(
      _tc1, out_shape=(jax.ShapeDtypeStruct((NP, H), f32),
                       jax.ShapeDtypeStruct((NP // 128, 128), f32),
                       jax.ShapeDtypeStruct((NP // 128, 128), f32)))(
          x, W1, w0_1q, w0_2q)
  sw1 = sw1q.reshape(NP, 1)
  sw2 = sw2q.reshape(NP, 1)
  agg1p, disb = _sc_fused_layer1(row2d, col2d, h1, zeros16)
  h2, h2p = pl.pallas_call(
      _tc3, out_shape=(jax.ShapeDtypeStruct((NP, F2), f32),
                       jax.ShapeDtypeStruct((NP, F2), f32)))(
          disb, agg1p, h1, sw1, b1r, W2p)
  agg2p = _sc_agg(row2d, col2d, h2p, zeros48, F2)
  out = pl.pallas_call(
      _tc4, out_shape=jax.ShapeDtypeStruct((N, C), f32))(
          disb, agg2p, h2, sw2, b2r)
  return out


# speedup vs baseline: 1.7524x; 1.7524x over previous
import functools

import jax
import jax.numpy as jnp
from jax import lax
from jax.experimental import pallas as pl
from jax.experimental.pallas import tpu as pltpu
from jax.experimental.pallas import tpu_sc as plsc

N = 10000
E = 320000
D_IN = 128
H = 16
C = 40

NP = 10240
F2 = 48
E2 = 327680

NC = 2
NS = 16
NW = NC * NS
EPW = E2 // NW
K = 128
CH = EPW // K
CHD = (E2 // NS) // K
CHQ = CH // 4
RP = NP // NS
DEG_Q = 16

_mesh = plsc.VectorSubcoreMesh(core_axis_name="c", subcore_axis_name="s")
_sc_params = pltpu.CompilerParams(use_tc_tiling_on_sc=False,
                                  needs_layout_passes=False)


def _agg_pipeline(tab, idxr_v, idxc_v, buf, acc_sp, semg, sems):
  def g(j, q):
    pltpu.async_copy(tab.at[idxr_v.at[j]], buf.at[q], semg[q])

  def sct(j, p):
    pltpu.async_copy(buf.at[p], acc_sp.at[idxc_v.at[j]], sems[p], add=True)

  def wait_g(q):
    pltpu.make_async_copy(tab.at[pl.ds(0, K)], buf.at[q], semg[q]).wait()

  def wait_s(p):
    pltpu.make_async_copy(buf.at[p], acc_sp.at[pl.ds(0, K)], sems[p]).wait()

  g(0, 0)
  g(1, 1)
  wait_g(0); sct(0, 0); g(2, 2)
  wait_g(1); sct(1, 1); g(3, 3)
  wait_g(2); sct(2, 2); wait_s(0); g(4, 0)
  wait_g(3); sct(3, 3); wait_s(1); g(5, 1)

  def body(t, _):
    for p in range(4):
      j = 4 * t + p
      q = (p + 2) % 4
      wait_g(p)
      sct(j, p)
      wait_s(q)
      g(j + 2, q)
    return 0

  lax.fori_loop(1, CHQ - 1, body, 0)

  for p in range(4):
    j = 4 * (CHQ - 1) + p
    q = (p + 2) % 4
    wait_g(p)
    sct(j, p)
    wait_s(q)
    if j + 2 < CH:
      g(j + 2, q)
  wait_s(2)
  wait_s(3)


def _sc_fused_layer1(row2d, col2d, h1, zeros16):
  @functools.partial(
      pl.kernel,
      out_type=(jax.ShapeDtypeStruct((NC, NP, H), jnp.float32),
                jax.ShapeDtypeStruct((NC, NP, H), jnp.float32)),
      mesh=_mesh,
      compiler_params=_sc_params,
      scratch_types=[
          pltpu.VMEM((CHD, K), jnp.int32),
          pltpu.VMEM((CH, K), jnp.int32),
          pltpu.VMEM((CH, K), jnp.int32),
          pltpu.VMEM((4, K, H), jnp.float32),
          pltpu.VMEM((RP, H), jnp.float32),
          pltpu.VMEM((RP, H), jnp.float32),
          pltpu.VMEM((RP, H), jnp.float32),
          pltpu.VMEM((RP, H), jnp.float32),
          pltpu.VMEM((RP,), jnp.float32),
          pltpu.VMEM((RP, H), jnp.float32),
          pltpu.VMEM((K, H), jnp.float32),
          pltpu.VMEM_SHARED((NP, H), jnp.float32),
          pltpu.VMEM_SHARED((NP, H), jnp.float32),
      ] + [pltpu.SemaphoreType.DMA] * 13,
  )
  def k(row_hbm, col_hbm, h1_hbm, z_hbm, agg_out, dis_out,
        idxd, idxr_v, idxc_v, buf, degst, h1v, h1pv, disrow, disf,
        zerov, onesv, acc_sp, h1p_sp, *sems13):
    si0, si1, si2, si3, semd = sems13[0:5]
    semg = sems13[5:9]
    sems = sems13[9:13]
    c = lax.axis_index("c")
    s = lax.axis_index("s")
    w = s * NC + c

    ld0 = pltpu.async_copy(col_hbm.at[pl.ds(s * CHD, CHD)], idxd, si0)
    ld1 = pltpu.async_copy(row_hbm.at[pl.ds(w * CH, CH)], idxr_v, si1)
    ld2 = pltpu.async_copy(col_hbm.at[pl.ds(w * CH, CH)], idxc_v, si2)
    ld3 = pltpu.async_copy(h1_hbm.at[pl.ds(s * RP, RP)], h1v, si3)

    def fill_ones(i, _):
      onesv[i, :] = jnp.ones((H,), jnp.float32)
      return 0

    lax.fori_loop(0, K, fill_ones, 0)
    pltpu.sync_copy(z_hbm.at[pl.ds(s * RP, RP)], zerov)
    pltpu.sync_copy(zerov, acc_sp.at[pl.ds(s * RP, RP)])
    ld0.wait()
    plsc.subcore_barrier()

    def wait_one_deg():
      pltpu.make_async_copy(onesv, acc_sp.at[pl.ds(0, K)], semd).wait()

    def dbody(j, _):
      pltpu.async_copy(onesv, acc_sp.at[idxd.at[j]], semd, add=True)

      @pl.when(j >= DEG_Q)
      def _():
        wait_one_deg()

      return 0

    lax.fori_loop(0, CHD, dbody, 0)

    def drain(j, _):
      wait_one_deg()
      return 0

    lax.fori_loop(0, DEG_Q, drain, 0)
    plsc.subcore_barrier()

    pltpu.sync_copy(acc_sp.at[pl.ds(s * RP, RP)], degst)
    iota = lax.iota(jnp.int32, 16)
    zi = jnp.zeros((16,), jnp.int32)

    def disbody(i, _):
      v = plsc.load_gather(degst, [i * 16 + iota, zi])
      bits = lax.bitcast_convert_type(v, jnp.int32)
      y = lax.bitcast_convert_type(jnp.int32(0x5F3759DF) - (bits >> 1),
                                   jnp.float32)
      for _ in range(4):
        y = y * (1.5 - 0.5 * v * y * y)
      y = jnp.where(v > 0, y, 0.0)
      disf[pl.ds(i * 16, 16)] = y
      return 0

    lax.fori_loop(0, RP // 16, disbody, 0)

    ld3.wait()

    def scale(r, _):
      db = plsc.load_gather(disf, [jnp.full((16,), r, jnp.int32)])
      h1pv[r, :] = h1v[r, :] * db
      disrow[r, :] = db
      return 0

    lax.fori_loop(0, RP, scale, 0)
    pltpu.sync_copy(h1pv, h1p_sp.at[pl.ds(s * RP, RP)])
    pltpu.sync_copy(disrow, dis_out.at[c, pl.ds(s * RP, RP)])
    ld1.wait()
    ld2.wait()
    plsc.subcore_barrier()
    pltpu.sync_copy(zerov, acc_sp.at[pl.ds(s * RP, RP)])
    plsc.subcore_barrier()

    _agg_pipeline(h1p_sp, idxr_v, idxc_v, buf, acc_sp, semg, sems)
    plsc.subcore_barrier()
    pltpu.sync_copy(acc_sp.at[pl.ds(s * RP, RP)],
                    agg_out.at[c, pl.ds(s * RP, RP)])

  return k(row2d, col2d, h1, zeros16)


def _sc_agg(row2d, col2d, table, zeros, f):
  @functools.partial(
      pl.kernel,
      out_type=jax.ShapeDtypeStruct((NC, NP, f), jnp.float32),
      mesh=_mesh,
      compiler_params=_sc_params,
      scratch_types=[
          pltpu.VMEM((CH, K), jnp.int32),
          pltpu.VMEM((CH, K), jnp.int32),
          pltpu.VMEM((4, K, f), jnp.float32),
          pltpu.VMEM((RP, f), jnp.float32),
          pltpu.VMEM_SHARED((NP, f), jnp.float32),
      ] + [pltpu.SemaphoreType.DMA] * 8,
  )
  def k(row_hbm, col_hbm, tab_hbm, z_hbm, out_hbm,
        idxr_v, idxc_v, buf, zero_v, acc_sp, *sems8):
    semg = sems8[0:4]
    sems = sems8[4:8]
    c = lax.axis_index("c")
    s = lax.axis_index("s")
    w = s * NC + c

    ldr = pltpu.async_copy(row_hbm.at[pl.ds(w * CH, CH)], idxr_v, semg[0])
    ldc = pltpu.async_copy(col_hbm.at[pl.ds(w * CH, CH)], idxc_v, semg[1])
    pltpu.sync_copy(z_hbm.at[pl.ds(s * RP, RP)], zero_v)
    pltpu.sync_copy(zero_v, acc_sp.at[pl.ds(s * RP, RP)])
    ldr.wait()
    ldc.wait()
    plsc.subcore_barrier()

    _agg_pipeline(tab_hbm, idxr_v, idxc_v, buf, acc_sp, semg, sems)
    plsc.subcore_barrier()
    pltpu.sync_copy(acc_sp.at[pl.ds(s * RP, RP)],
                    out_hbm.at[c, pl.ds(s * RP, RP)])

  return k(row2d, col2d, table, zeros)


def _tc1(x_ref, w1_ref, w01_ref, w02_ref, h1_ref, sw1_ref, sw2_ref):
  h1_ref[pl.ds(0, N), :] = jnp.dot(x_ref[...], w1_ref[...],
                                   preferred_element_type=jnp.float32)
  h1_ref[pl.ds(N, NP - N), :] = jnp.zeros((NP - N, H), jnp.float32)
  sw1_ref[...] = jax.nn.sigmoid(w01_ref[...])
  sw2_ref[...] = jax.nn.sigmoid(w02_ref[...])


def _tc3(disb_ref, aggp_ref, h1_ref, sw_ref, b1_ref, w2_ref,
         h2_ref, h2p_ref):
  dis16 = disb_ref[0]
  agg = aggp_ref[0] + aggp_ref[1]
  z = dis16 * agg + sw_ref[...] * h1_ref[...] + b1_ref[...]
  z = jnp.maximum(z, 0.0)
  h2 = jnp.dot(z, w2_ref[...], preferred_element_type=jnp.float32)
  h2_ref[...] = h2
  h2p_ref[...] = h2 * dis16[:, 0:1]


def _tc4(disb_ref, aggp_ref, h2_ref, sw_ref, b2_ref, o_ref):
  dis1 = disb_ref[0][:, 0:1]
  agg = aggp_ref[0] + aggp_ref[1]
  out = dis1 * agg + sw_ref[...] * h2_ref[...] + b2_ref[...]
  o_ref[...] = out[:N, :C]


def kernel(x, edge_index, w0_1, W1, b1, w0_2, W2, b2):
  ei = edge_index.astype(jnp.int32)
  pad_idx = (jnp.arange(E2 - E, dtype=jnp.int32) % (NP - N)) + N
  eip = jnp.concatenate([ei, jnp.stack([pad_idx, pad_idx])], axis=1)
  row2d = eip[0].reshape(NW * CH, K)
  col2d = eip[1].reshape(NW * CH, K)

  w0_1q = jnp.pad(w0_1, (0, NP - N)).reshape(NP // 128, 128)
  w0_2q = jnp.pad(w0_2, (0, NP - N)).reshape(NP // 128, 128)
  W2p = jnp.pad(W2, ((0, 0), (0, F2 - C)))
  b1r = b1.reshape(1, H)
  b2r = jnp.pad(b2, (0, F2 - C)).reshape(1, F2)
  zeros48 = jnp.zeros((NP, F2), jnp.float32)
  zeros16 = jnp.zeros((NP, H), jnp.float32)

  f32 = jnp.float32
  h1, sw1q, sw2q = pl.pallas_call(
      _tc1, out_shape=(jax.ShapeDtypeStruct((NP, H), f32),
                       jax.ShapeDtypeStruct((NP // 128, 128), f32),
                       jax.ShapeDtypeStruct((NP // 128, 128), f32)))(
          x, W1, w0_1q, w0_2q)
  sw1 = sw1q.reshape(NP, 1)
  sw2 = sw2q.reshape(NP, 1)
  agg1p, disb = _sc_fused_layer1(row2d, col2d, h1, zeros16)
  h2, h2p = pl.pallas_call(
      _tc3, out_shape=(jax.ShapeDtypeStruct((NP, F2), f32),
                       jax.ShapeDtypeStruct((NP, F2), f32)))(
          disb, agg1p, h1, sw1, b1r, W2p)
  agg2p = _sc_agg(row2d, col2d, h2p, zeros48, F2)
  out = pl.pallas_call(
      _tc4, out_shape=jax.ShapeDtypeStruct((N, C), f32))(
          disb, agg2p, h2, sw2, b2r)
  return out


# speedup vs baseline: 1.9445x; 1.1096x over previous
import functools

import jax
import jax.numpy as jnp
from jax import lax
from jax.experimental import pallas as pl
from jax.experimental.pallas import tpu as pltpu
from jax.experimental.pallas import tpu_sc as plsc

N = 10000
E = 320000
D_IN = 128
H = 16
C = 40

NP = 10240
F2 = 48

NC = 2
NS = 16
NW = NC * NS
K = 128
NR = E // K
CH = 80
CHD = 160
NPAD = NW * CH - NR
CHQ = CH // 4
RP = NP // NS
HRP = RP // 2
DEG_Q = 16

PKH = NP * H // 128
PKF = F2 // H

_mesh = plsc.VectorSubcoreMesh(core_axis_name="c", subcore_axis_name="s")
_sc_params = pltpu.CompilerParams(use_tc_tiling_on_sc=False,
                                  needs_layout_passes=False)


def _agg_pipeline(tab, idxr_v, idxc_v, buf, acc_sp, semg, sems):
  def g(j, q):
    pltpu.async_copy(tab.at[idxr_v.at[j]], buf.at[q], semg[q])

  def sct(j, p):
    pltpu.async_copy(buf.at[p], acc_sp.at[idxc_v.at[j]], sems[p], add=True)

  def wait_g(q):
    pltpu.make_async_copy(tab.at[pl.ds(0, K)], buf.at[q], semg[q]).wait()

  def wait_s(p):
    pltpu.make_async_copy(buf.at[p], acc_sp.at[pl.ds(0, K)], sems[p]).wait()

  g(0, 0)
  g(1, 1)
  wait_g(0); sct(0, 0); g(2, 2)
  wait_g(1); sct(1, 1); g(3, 3)
  wait_g(2); sct(2, 2); wait_s(0); g(4, 0)
  wait_g(3); sct(3, 3); wait_s(1); g(5, 1)

  def body(t, _):
    for p in range(4):
      j = 4 * t + p
      q = (p + 2) % 4
      wait_g(p)
      sct(j, p)
      wait_s(q)
      g(j + 2, q)
    return 0

  lax.fori_loop(1, CHQ - 1, body, 0)

  for p in range(4):
    j = 4 * (CHQ - 1) + p
    q = (p + 2) % 4
    wait_g(p)
    sct(j, p)
    wait_s(q)
    if j + 2 < CH:
      g(j + 2, q)
  wait_s(2)
  wait_s(3)


def _load_agg_idx(dst, real, pad, w, sem):
  @pl.when(w < NW - 1)
  def _():
    pltpu.async_copy(real.at[pl.ds(w * CH, CH)], dst, sem)

  @pl.when(w == NW - 1)
  def _():
    nreal = NR - (NW - 1) * CH
    pltpu.async_copy(real.at[pl.ds((NW - 1) * CH, nreal)],
                     dst.at[pl.ds(0, nreal)], sem)
    pltpu.async_copy(pad.at[pl.ds(0, NPAD)],
                     dst.at[pl.ds(nreal, NPAD)], sem)


def _sigmoid16(src_v, dst_v):
  def body(i, _):
    v = src_v[pl.ds(i * 16, 16)]
    dst_v[pl.ds(i * 16, 16)] = 1.0 / (1.0 + jnp.exp(-v))
    return 0

  lax.fori_loop(0, RP // 16, body, 0)


def _sc_fused_layer1(row2d, col2d, pad2d, h1, w01, zeros16):
  @functools.partial(
      pl.kernel,
      out_type=(jax.ShapeDtypeStruct((NC, NP, H), jnp.float32),
                jax.ShapeDtypeStruct((NP, H), jnp.float32),
                jax.ShapeDtypeStruct((NP, F2), jnp.float32),
                jax.ShapeDtypeStruct((NP, H), jnp.float32)),
      mesh=_mesh,
      compiler_params=_sc_params,
      scratch_types=[
          pltpu.VMEM((CHD, K), jnp.int32),
          pltpu.VMEM((CH, K), jnp.int32),
          pltpu.VMEM((CH, K), jnp.int32),
          pltpu.VMEM((4, K, H), jnp.float32),
          pltpu.VMEM((RP, H), jnp.float32),
          pltpu.VMEM((RP, H), jnp.float32),
          pltpu.VMEM((RP, H), jnp.float32),
          pltpu.VMEM((RP, H), jnp.float32),
          pltpu.VMEM((RP, H), jnp.float32),
          pltpu.VMEM((RP,), jnp.float32),
          pltpu.VMEM((RP,), jnp.float32),
          pltpu.VMEM((RP,), jnp.float32),
          pltpu.VMEM((K, H), jnp.float32),
          pltpu.VMEM_SHARED((NP, H), jnp.float32),
          pltpu.VMEM_SHARED((NP, H), jnp.float32),
      ] + [pltpu.SemaphoreType.DMA] * 14,
  )
  def k(row_hbm, col_hbm, pad_hbm, h1_hbm, w0_hbm, z_hbm,
        agg_out, dis16_out, dis48_out, sw16_out,
        idxd, idxr_v, idxc_v, buf, degst, h1v, h1pv, disrow, swrow,
        disf, w0v, swf, onesv, acc_sp, h1p_sp, *sems14):
    si0, si1, si2, si3, si4, semd = sems14[0:6]
    semg = sems14[6:10]
    sems = sems14[10:14]
    c = lax.axis_index("c")
    s = lax.axis_index("s")
    w = s * NC + c

    @pl.when(s < NS - 1)
    def _():
      pltpu.async_copy(col_hbm.at[pl.ds(s * CHD, CHD)], idxd, si0)

    @pl.when(s == NS - 1)
    def _():
      nreal = NR - (NS - 1) * CHD
      pltpu.async_copy(col_hbm.at[pl.ds((NS - 1) * CHD, nreal)],
                       idxd.at[pl.ds(0, nreal)], si0)
      pltpu.async_copy(pad_hbm.at[pl.ds(0, NPAD)],
                       idxd.at[pl.ds(nreal, NPAD)], si0)

    _load_agg_idx(idxr_v, row_hbm, pad_hbm, w, si1)
    _load_agg_idx(idxc_v, col_hbm, pad_hbm, w, si2)
    ld3 = pltpu.async_copy(h1_hbm.at[pl.ds(s * RP, RP)], h1v, si3)
    ld4 = pltpu.async_copy(w0_hbm.at[pl.ds(s * RP, RP)], w0v, si4)

    def fill_ones(i, _):
      onesv[i, :] = jnp.ones((H,), jnp.float32)
      return 0

    lax.fori_loop(0, K, fill_ones, 0)
    pltpu.sync_copy(z_hbm.at[pl.ds(s * RP, RP)], degst)
    pltpu.sync_copy(degst, acc_sp.at[pl.ds(s * RP, RP)])
    pltpu.make_async_copy(col_hbm.at[pl.ds(0, CHD)], idxd, si0).wait()
    plsc.subcore_barrier()

    def wait_one_deg():
      pltpu.make_async_copy(onesv, acc_sp.at[pl.ds(0, K)], semd).wait()

    def dbody(j, _):
      pltpu.async_copy(onesv, acc_sp.at[idxd.at[j]], semd, add=True)

      @pl.when(j >= DEG_Q)
      def _():
        wait_one_deg()

      return 0

    lax.fori_loop(0, CHD, dbody, 0)

    def drain(j, _):
      wait_one_deg()
      return 0

    lax.fori_loop(0, DEG_Q, drain, 0)
    plsc.subcore_barrier()

    pltpu.sync_copy(acc_sp.at[pl.ds(s * RP, RP)], degst)
    iota = lax.iota(jnp.int32, 16)
    zi = jnp.zeros((16,), jnp.int32)

    def disbody(i, _):
      v = plsc.load_gather(degst, [i * 16 + iota, zi])
      bits = lax.bitcast_convert_type(v, jnp.int32)
      y = lax.bitcast_convert_type(jnp.int32(0x5F3759DF) - (bits >> 1),
                                   jnp.float32)
      for _ in range(4):
        y = y * (1.5 - 0.5 * v * y * y)
      y = jnp.where(v > 0, y, 0.0)
      disf[pl.ds(i * 16, 16)] = y
      return 0

    lax.fori_loop(0, RP // 16, disbody, 0)

    ld4.wait()
    _sigmoid16(w0v, swf)

    ld3.wait()

    def scale(r, _):
      db = plsc.load_gather(disf, [jnp.full((16,), r, jnp.int32)])
      sb = plsc.load_gather(swf, [jnp.full((16,), r, jnp.int32)])
      h1pv[r, :] = h1v[r, :] * db
      disrow[r, :] = db
      swrow[r, :] = sb
      return 0

    lax.fori_loop(0, RP, scale, 0)
    pltpu.sync_copy(h1pv, h1p_sp.at[pl.ds(s * RP, RP)])
    half = s * RP + c * HRP
    pltpu.sync_copy(disrow.at[pl.ds(c * HRP, HRP)],
                    dis16_out.at[pl.ds(half, HRP)])
    pltpu.sync_copy(swrow.at[pl.ds(c * HRP, HRP)],
                    sw16_out.at[pl.ds(half, HRP)])
    for b in range(PKF):
      pltpu.sync_copy(disrow.at[pl.ds(c * HRP, HRP)],
                      dis48_out.at[pl.ds(half, HRP), pl.ds(b * H, H)])
    pltpu.make_async_copy(row_hbm.at[pl.ds(0, CH)], idxr_v, si1).wait()
    pltpu.make_async_copy(col_hbm.at[pl.ds(0, CH)], idxc_v, si2).wait()
    plsc.subcore_barrier()
    pltpu.sync_copy(z_hbm.at[pl.ds(s * RP, RP)], degst)
    pltpu.sync_copy(degst, acc_sp.at[pl.ds(s * RP, RP)])
    plsc.subcore_barrier()

    _agg_pipeline(h1p_sp, idxr_v, idxc_v, buf, acc_sp, semg, sems)
    plsc.subcore_barrier()
    pltpu.sync_copy(acc_sp.at[pl.ds(s * RP, RP)],
                    agg_out.at[c, pl.ds(s * RP, RP)])

  return k(row2d, col2d, pad2d, h1, w01, zeros16)


def _sc_agg2(row2d, col2d, pad2d, table, h2, w02, zeros48):
  @functools.partial(
      pl.kernel,
      out_type=(jax.ShapeDtypeStruct((NC, NP, F2), jnp.float32),
                jax.ShapeDtypeStruct((NP, F2), jnp.float32)),
      mesh=_mesh,
      compiler_params=_sc_params,
      scratch_types=[
          pltpu.VMEM((CH, K), jnp.int32),
          pltpu.VMEM((CH, K), jnp.int32),
          pltpu.VMEM((4, K, F2), jnp.float32),
          pltpu.VMEM((RP, F2), jnp.float32),
          pltpu.VMEM((RP,), jnp.float32),
          pltpu.VMEM((RP,), jnp.float32),
          pltpu.VMEM_SHARED((NP, F2), jnp.float32),
      ] + [pltpu.SemaphoreType.DMA] * 11,
  )
  def k(row_hbm, col_hbm, pad_hbm, tab_hbm, h2_hbm, w0_hbm, z_hbm,
        out_hbm, hsl_out,
        idxr_v, idxc_v, buf, h2v, w0v, swf, acc_sp, *sems11):
    si1, si2, si4 = sems11[0:3]
    semg = sems11[3:7]
    sems = sems11[7:11]
    c = lax.axis_index("c")
    s = lax.axis_index("s")
    w = s * NC + c

    _load_agg_idx(idxr_v, row_hbm, pad_hbm, w, si1)
    _load_agg_idx(idxc_v, col_hbm, pad_hbm, w, si2)
    ld4 = pltpu.async_copy(w0_hbm.at[pl.ds(s * RP, RP)], w0v, si4)
    pltpu.sync_copy(z_hbm.at[pl.ds(s * RP, RP)], h2v)
    pltpu.sync_copy(h2v, acc_sp.at[pl.ds(s * RP, RP)])
    half = s * RP + c * HRP
    pltpu.sync_copy(h2_hbm.at[pl.ds(half, HRP)], h2v.at[pl.ds(0, HRP)])
    ld4.wait()
    _sigmoid16(w0v, swf)

    def hsl(r, _):
      sb = plsc.load_gather(swf, [jnp.full((16,), r, jnp.int32) + c * HRP])
      for b in range(PKF):
        h2v[r, pl.ds(b * H, H)] = h2v[r, pl.ds(b * H, H)] * sb
      return 0

    lax.fori_loop(0, HRP, hsl, 0)
    pltpu.sync_copy(h2v.at[pl.ds(0, HRP)], hsl_out.at[pl.ds(half, HRP)])
    pltpu.make_async_copy(row_hbm.at[pl.ds(0, CH)], idxr_v, si1).wait()
    pltpu.make_async_copy(col_hbm.at[pl.ds(0, CH)], idxc_v, si2).wait()
    plsc.subcore_barrier()

    _agg_pipeline(tab_hbm, idxr_v, idxc_v, buf, acc_sp, semg, sems)
    plsc.subcore_barrier()
    pltpu.sync_copy(acc_sp.at[pl.ds(s * RP, RP)],
                    out_hbm.at[c, pl.ds(s * RP, RP)])

  return k(row2d, col2d, pad2d, table, h2, w02, zeros48)


def _tc1(xr_ref, w1b_ref, h1_ref):
  h1_ref[pl.ds(0, N // 8), :] = jnp.dot(xr_ref[...], w1b_ref[...],
                                        preferred_element_type=jnp.float32)
  h1_ref[pl.ds(N // 8, (NP - N) // 8), :] = jnp.zeros(((NP - N) // 8, 128),
                                                      jnp.float32)


def _tc3(dis16_ref, sw16_ref, dis48_ref, aggp_ref, h1_ref, b1_ref, w2b_ref,
         h2_ref, h2p_ref):
  agg = aggp_ref[0] + aggp_ref[1]
  z = dis16_ref[...] * agg + sw16_ref[...] * h1_ref[...] + b1_ref[...]
  z = jnp.maximum(z, 0.0)
  h2 = jnp.dot(z, w2b_ref[...], preferred_element_type=jnp.float32)
  h2_ref[...] = h2
  h2p_ref[...] = h2 * dis48_ref[...]


def _tc4(dis48_ref, aggp_ref, hsl2_ref, b2_ref, o_ref):
  agg = aggp_ref[0] + aggp_ref[1]
  o_ref[...] = dis48_ref[...] * agg + hsl2_ref[...] + b2_ref[...]


def kernel(x, edge_index, w0_1, W1, b1, w0_2, W2, b2):
  f32 = jnp.float32
  ei = edge_index.astype(jnp.int32)
  row2d = ei[0].reshape(NR, K)
  col2d = ei[1].reshape(NR, K)
  pad2d = (jnp.arange(NPAD * K, dtype=jnp.int32) % (NP - N) + N).reshape(
      NPAD, K)

  w0_1p = jnp.pad(w0_1, (0, NP - N))
  w0_2p = jnp.pad(w0_2, (0, NP - N))
  eye8 = jnp.eye(8, dtype=f32)
  W1b = (eye8[:, None, :, None] * W1[None, :, None, :]).reshape(8 * D_IN,
                                                                8 * H)
  W2p = jnp.pad(W2, ((0, 0), (0, F2 - C)))
  W2b = (eye8[:, None, :, None] * W2p[None, :, None, :]).reshape(8 * H,
                                                                 8 * F2)
  b1t = jnp.tile(b1, 8).reshape(1, 128)
  b2t = jnp.tile(jnp.pad(b2, (0, F2 - C)), 8).reshape(1, 8 * F2)
  zeros16 = jnp.zeros((NP, H), f32)
  zeros48 = jnp.zeros((NP, F2), f32)

  xr = x.reshape(N // 8, 8 * D_IN)
  h1pk = pl.pallas_call(
      _tc1, out_shape=jax.ShapeDtypeStruct((PKH, 128), f32))(xr, W1b)

  agg1p, dis16, dis48, sw16 = _sc_fused_layer1(
      row2d, col2d, pad2d, h1pk.reshape(NP, H), w0_1p, zeros16)

  h2pk, h2ppk = pl.pallas_call(
      _tc3, out_shape=(jax.ShapeDtypeStruct((PKH, 8 * F2), f32),
                       jax.ShapeDtypeStruct((PKH, 8 * F2), f32)))(
          dis16.reshape(PKH, 128), sw16.reshape(PKH, 128),
          dis48.reshape(PKH, 8 * F2), agg1p.reshape(NC, PKH, 128),
          h1pk, b1t, W2b)

  agg2p, hsl2 = _sc_agg2(row2d, col2d, pad2d, h2ppk.reshape(NP, F2),
                         h2pk.reshape(NP, F2), w0_2p, zeros48)

  outp = pl.pallas_call(
      _tc4, out_shape=jax.ShapeDtypeStruct((PKH, 8 * F2), f32))(
          dis48.reshape(PKH, 8 * F2), agg2p.reshape(NC, PKH, 8 * F2),
          hsl2.reshape(PKH, 8 * F2), b2t)
  return outp.reshape(NP, F2)[:N, :C]


# speedup vs baseline: 2.0788x; 1.0691x over previous
import functools

import jax
import jax.numpy as jnp
from jax import lax
from jax.experimental import pallas as pl
from jax.experimental.pallas import tpu as pltpu
from jax.experimental.pallas import tpu_sc as plsc

N = 10000
E = 320000
D_IN = 128
H = 16
C = 40

NP = 10240
F2 = 48

NC = 2
NS = 16
NW = NC * NS
K = 128
NR = E // K
CH = 80
CHD = 160
NPAD = NW * CH - NR
CHQ = CH // 4
RP = NP // NS
HRP = RP // 2
DEG_Q = 16

PKH = NP * H // 128
PKF = F2 // H

_mesh = plsc.VectorSubcoreMesh(core_axis_name="c", subcore_axis_name="s")
_sc_params = pltpu.CompilerParams(use_tc_tiling_on_sc=False,
                                  needs_layout_passes=False)


def _agg_pipeline(tab, idxr_v, idxc_v, buf, acc_sp, semg, sems):
  def g(j, q):
    pltpu.async_copy(tab.at[idxr_v.at[j]], buf.at[q], semg[q])

  def sct(j, p):
    pltpu.async_copy(buf.at[p], acc_sp.at[idxc_v.at[j]], sems[p], add=True)

  def wait_g(q):
    pltpu.make_async_copy(tab.at[pl.ds(0, K)], buf.at[q], semg[q]).wait()

  def wait_s(p):
    pltpu.make_async_copy(buf.at[p], acc_sp.at[pl.ds(0, K)], sems[p]).wait()

  g(0, 0)
  g(1, 1)
  wait_g(0); sct(0, 0); g(2, 2)
  wait_g(1); sct(1, 1); g(3, 3)
  wait_g(2); sct(2, 2); wait_s(0); g(4, 0)
  wait_g(3); sct(3, 3); wait_s(1); g(5, 1)

  def body(t, _):
    for p in range(4):
      j = 4 * t + p
      q = (p + 2) % 4
      wait_g(p)
      sct(j, p)
      wait_s(q)
      g(j + 2, q)
    return 0

  lax.fori_loop(1, CHQ - 1, body, 0)

  for p in range(4):
    j = 4 * (CHQ - 1) + p
    q = (p + 2) % 4
    wait_g(p)
    sct(j, p)
    wait_s(q)
    if j + 2 < CH:
      g(j + 2, q)
  wait_s(2)
  wait_s(3)


def _load_agg_idx(dst, ei3, d, pad, w, sem):
  @pl.when(w < NW - 1)
  def _():
    pltpu.async_copy(ei3.at[d, pl.ds(w * CH, CH)], dst, sem)

  @pl.when(w == NW - 1)
  def _():
    nreal = NR - (NW - 1) * CH
    pltpu.async_copy(ei3.at[d, pl.ds((NW - 1) * CH, nreal)],
                     dst.at[pl.ds(0, nreal)], sem)
    pltpu.async_copy(pad.at[pl.ds(0, NPAD)],
                     dst.at[pl.ds(nreal, NPAD)], sem)


def _sigmoid16(src_v, dst_v):
  def body(i, _):
    v = src_v[pl.ds(i * 16, 16)]
    dst_v[pl.ds(i * 16, 16)] = 1.0 / (1.0 + jnp.exp(-v))
    return 0

  lax.fori_loop(0, RP // 16, body, 0)


def _sc_fused_layer1(ei3, pad2d, h1, w01):
  @functools.partial(
      pl.kernel,
      out_type=(jax.ShapeDtypeStruct((NC, NP, H), jnp.float32),
                jax.ShapeDtypeStruct((PKH, 128), jnp.float32),
                jax.ShapeDtypeStruct((PKH, 128), jnp.float32)),
      mesh=_mesh,
      compiler_params=_sc_params,
      scratch_types=[
          pltpu.VMEM((CHD, K), jnp.int32),
          pltpu.VMEM((CH, K), jnp.int32),
          pltpu.VMEM((CH, K), jnp.int32),
          pltpu.VMEM((4, K, H), jnp.float32),
          pltpu.VMEM((RP, H), jnp.float32),
          pltpu.VMEM((RP // 8, 128), jnp.float32),
          pltpu.VMEM((RP, H), jnp.float32),
          pltpu.VMEM((RP // 8, 128), jnp.float32),
          pltpu.VMEM((RP // 8, 128), jnp.float32),
          pltpu.VMEM((RP,), jnp.float32),
          pltpu.VMEM((RP,), jnp.float32),
          pltpu.VMEM((RP,), jnp.float32),
          pltpu.VMEM((K, H), jnp.float32),
          pltpu.VMEM_SHARED((NP, H), jnp.float32),
          pltpu.VMEM_SHARED((NP, H), jnp.float32),
      ] + [pltpu.SemaphoreType.DMA] * 14,
  )
  def k(ei3_hbm, pad_hbm, h1_hbm, w0_hbm,
        agg_out, dis16_out, sw16_out,
        idxd, idxr_v, idxc_v, buf, degst, h1v, h1pv, disrow, swrow,
        disf, w0v, swf, onesv, acc_sp, h1p_sp, *sems14):
    si0, si1, si2, si3, si4, semd = sems14[0:6]
    semg = sems14[6:10]
    sems = sems14[10:14]
    c = lax.axis_index("c")
    s = lax.axis_index("s")
    w = s * NC + c

    @pl.when(s < NS - 1)
    def _():
      pltpu.async_copy(ei3_hbm.at[1, pl.ds(s * CHD, CHD)], idxd, si0)

    @pl.when(s == NS - 1)
    def _():
      nreal = NR - (NS - 1) * CHD
      pltpu.async_copy(ei3_hbm.at[1, pl.ds((NS - 1) * CHD, nreal)],
                       idxd.at[pl.ds(0, nreal)], si0)
      pltpu.async_copy(pad_hbm.at[pl.ds(0, NPAD)],
                       idxd.at[pl.ds(nreal, NPAD)], si0)

    _load_agg_idx(idxr_v, ei3_hbm, 0, pad_hbm, w, si1)
    _load_agg_idx(idxc_v, ei3_hbm, 1, pad_hbm, w, si2)
    ld3 = pltpu.async_copy(h1_hbm.at[pl.ds(s * (RP // 8), RP // 8)], h1v, si3)
    ld4 = pltpu.async_copy(w0_hbm.at[pl.ds(s * RP, RP)], w0v, si4)

    def fill_ones(i, _):
      onesv[i, :] = jnp.ones((H,), jnp.float32)
      return 0

    lax.fori_loop(0, K, fill_ones, 0)

    def fill_zero(i, _):
      degst[i, :] = jnp.zeros((H,), jnp.float32)
      return 0

    lax.fori_loop(0, RP, fill_zero, 0)
    pltpu.sync_copy(degst, acc_sp.at[pl.ds(s * RP, RP)])
    pltpu.make_async_copy(ei3_hbm.at[1, pl.ds(0, CHD)], idxd, si0).wait()
    plsc.subcore_barrier()

    def wait_one_deg():
      pltpu.make_async_copy(onesv, acc_sp.at[pl.ds(0, K)], semd).wait()

    def dbody(j, _):
      pltpu.async_copy(onesv, acc_sp.at[idxd.at[j]], semd, add=True)

      @pl.when(j >= DEG_Q)
      def _():
        wait_one_deg()

      return 0

    lax.fori_loop(0, CHD, dbody, 0)

    def drain(j, _):
      wait_one_deg()
      return 0

    lax.fori_loop(0, DEG_Q, drain, 0)
    plsc.subcore_barrier()

    pltpu.sync_copy(acc_sp.at[pl.ds(s * RP, RP)], degst)
    iota = lax.iota(jnp.int32, 16)
    zi = jnp.zeros((16,), jnp.int32)

    def disbody(i, _):
      v = plsc.load_gather(degst, [i * 16 + iota, zi])
      bits = lax.bitcast_convert_type(v, jnp.int32)
      y = lax.bitcast_convert_type(jnp.int32(0x5F3759DF) - (bits >> 1),
                                   jnp.float32)
      for _ in range(4):
        y = y * (1.5 - 0.5 * v * y * y)
      y = jnp.where(v > 0, y, 0.0)
      disf[pl.ds(i * 16, 16)] = y
      return 0

    lax.fori_loop(0, RP // 16, disbody, 0)

    ld4.wait()
    _sigmoid16(w0v, swf)

    ld3.wait()

    def scale(ri, _):
      for a in range(8):
        r = 8 * ri + a
        db = plsc.load_gather(disf, [jnp.full((16,), r, jnp.int32)])
        sb = plsc.load_gather(swf, [jnp.full((16,), r, jnp.int32)])
        h1pv[r, :] = h1v[ri, pl.ds(a * H, H)] * db
        disrow[ri, pl.ds(a * H, H)] = db
        swrow[ri, pl.ds(a * H, H)] = sb
      return 0

    lax.fori_loop(0, RP // 8, scale, 0)
    pltpu.sync_copy(h1pv, h1p_sp.at[pl.ds(s * RP, RP)])
    half = s * RP + c * HRP
    phalf = s * (RP // 8) + c * (HRP // 8)
    pltpu.sync_copy(disrow.at[pl.ds(c * (HRP // 8), HRP // 8)],
                    dis16_out.at[pl.ds(phalf, HRP // 8)])
    pltpu.sync_copy(swrow.at[pl.ds(c * (HRP // 8), HRP // 8)],
                    sw16_out.at[pl.ds(phalf, HRP // 8)])
    pltpu.make_async_copy(ei3_hbm.at[0, pl.ds(0, CH)], idxr_v, si1).wait()
    pltpu.make_async_copy(ei3_hbm.at[1, pl.ds(0, CH)], idxc_v, si2).wait()
    plsc.subcore_barrier()
    def fill_zero2(i, _):
      degst[i, :] = jnp.zeros((H,), jnp.float32)
      return 0

    lax.fori_loop(0, RP, fill_zero2, 0)
    pltpu.sync_copy(degst, acc_sp.at[pl.ds(s * RP, RP)])
    plsc.subcore_barrier()

    _agg_pipeline(h1p_sp, idxr_v, idxc_v, buf, acc_sp, semg, sems)
    plsc.subcore_barrier()
    pltpu.sync_copy(acc_sp.at[pl.ds(s * RP, RP)],
                    agg_out.at[c, pl.ds(s * RP, RP)])

  return k(ei3, pad2d, h1, w01)


PKQ = NP * F2 // 384


def _sc_agg2(ei3, pad2d, table, h2pk, w02, zeros48):
  @functools.partial(
      pl.kernel,
      out_type=(jax.ShapeDtypeStruct((NC, NP, F2), jnp.float32),
                jax.ShapeDtypeStruct((PKQ, 384), jnp.float32)),
      mesh=_mesh,
      compiler_params=_sc_params,
      scratch_types=[
          pltpu.VMEM((CH, K), jnp.int32),
          pltpu.VMEM((CH, K), jnp.int32),
          pltpu.VMEM((4, K, F2), jnp.float32),
          pltpu.VMEM((RP, F2), jnp.float32),
          pltpu.VMEM((HRP // 8, 384), jnp.float32),
          pltpu.VMEM((RP,), jnp.float32),
          pltpu.VMEM((RP,), jnp.float32),
          pltpu.VMEM_SHARED((NP, F2), jnp.float32),
      ] + [pltpu.SemaphoreType.DMA] * 11,
  )
  def k(ei3_hbm, pad_hbm, tab_hbm, h2_hbm, w0_hbm, z_hbm,
        out_hbm, hsl_out,
        idxr_v, idxc_v, buf, zerov, h2v, w0v, swf, acc_sp, *sems11):
    si1, si2, si4 = sems11[0:3]
    semg = sems11[3:7]
    sems = sems11[7:11]
    c = lax.axis_index("c")
    s = lax.axis_index("s")
    w = s * NC + c

    _load_agg_idx(idxr_v, ei3_hbm, 0, pad_hbm, w, si1)
    _load_agg_idx(idxc_v, ei3_hbm, 1, pad_hbm, w, si2)
    ld4 = pltpu.async_copy(w0_hbm.at[pl.ds(s * RP, RP)], w0v, si4)
    phalf = s * (RP // 8) + c * (HRP // 8)
    ld5 = pltpu.async_copy(h2_hbm.at[pl.ds(phalf, HRP // 8)], h2v, si4)
    pltpu.sync_copy(z_hbm.at[pl.ds(s * RP, RP)], zerov)
    pltpu.sync_copy(zerov, acc_sp.at[pl.ds(s * RP, RP)])
    ld4.wait()
    ld5.wait()
    _sigmoid16(w0v, swf)

    def hsl(ri, _):
      for a in range(8):
        r = 8 * ri + a + c * HRP
        sb = plsc.load_gather(swf, [jnp.full((16,), r, jnp.int32)])
        for b in range(PKF):
          off = a * F2 + b * H
          h2v[ri, pl.ds(off, H)] = h2v[ri, pl.ds(off, H)] * sb
      return 0

    lax.fori_loop(0, HRP // 8, hsl, 0)
    pltpu.sync_copy(h2v, hsl_out.at[pl.ds(phalf, HRP // 8)])
    pltpu.make_async_copy(ei3_hbm.at[0, pl.ds(0, CH)], idxr_v, si1).wait()
    pltpu.make_async_copy(ei3_hbm.at[1, pl.ds(0, CH)], idxc_v, si2).wait()
    plsc.subcore_barrier()

    _agg_pipeline(tab_hbm, idxr_v, idxc_v, buf, acc_sp, semg, sems)
    plsc.subcore_barrier()
    pltpu.sync_copy(acc_sp.at[pl.ds(s * RP, RP)],
                    out_hbm.at[c, pl.ds(s * RP, RP)])

  return k(ei3, pad2d, table, h2pk, w02, zeros48)


def _tc1(xr_ref, w1b_ref, h1_ref):
  h1_ref[pl.ds(0, N // 8), :] = jnp.dot(xr_ref[...], w1b_ref[...],
                                        preferred_element_type=jnp.float32)
  h1_ref[pl.ds(N // 8, (NP - N) // 8), :] = jnp.zeros(((NP - N) // 8, 128),
                                                      jnp.float32)


def _dis48_from_packed(dis16):
  return jnp.concatenate(
      [jnp.broadcast_to(dis16[:, 16 * a:16 * a + 1], (PKH, F2))
       for a in range(8)], axis=1)


def _tc3(dis16_ref, sw16_ref, aggp_ref, h1_ref, b1_ref, w2b_ref,
         h2_ref, h2p_ref):
  agg = aggp_ref[0] + aggp_ref[1]
  dis16 = dis16_ref[...]
  z = dis16 * agg + sw16_ref[...] * h1_ref[...] + b1_ref[...]
  z = jnp.maximum(z, 0.0)
  h2 = jnp.dot(z, w2b_ref[...], preferred_element_type=jnp.float32)
  h2_ref[...] = h2
  h2p_ref[...] = h2 * _dis48_from_packed(dis16)


def _tc4(dis16_ref, aggp_ref, hsl2_ref, b2_ref, o_ref):
  agg = aggp_ref[0] + aggp_ref[1]
  o_ref[...] = (_dis48_from_packed(dis16_ref[...]) * agg + hsl2_ref[...]
                + b2_ref[...])


def kernel(x, edge_index, w0_1, W1, b1, w0_2, W2, b2):
  f32 = jnp.float32
  ei3 = edge_index.astype(jnp.int32).reshape(2, NR, K)
  pad2d = (jnp.arange(NPAD * K, dtype=jnp.int32) % (NP - N) + N).reshape(
      NPAD, K)

  w0_1p = jnp.pad(w0_1, (0, NP - N))
  w0_2p = jnp.pad(w0_2, (0, NP - N))
  eye8 = jnp.eye(8, dtype=f32)
  W1b = (eye8[:, None, :, None] * W1[None, :, None, :]).reshape(8 * D_IN,
                                                                8 * H)
  W2p = jnp.pad(W2, ((0, 0), (0, F2 - C)))
  W2b = (eye8[:, None, :, None] * W2p[None, :, None, :]).reshape(8 * H,
                                                                 8 * F2)
  b1t = jnp.tile(b1, 8).reshape(1, 128)
  b2t = jnp.tile(jnp.pad(b2, (0, F2 - C)), 8).reshape(1, 8 * F2)
  zeros48 = jnp.zeros((NP, F2), f32)

  xr = x.reshape(N // 8, 8 * D_IN)
  h1pk = pl.pallas_call(
      _tc1, out_shape=jax.ShapeDtypeStruct((PKH, 128), f32))(xr, W1b)

  agg1p, dis16pk, sw16pk = _sc_fused_layer1(
      ei3, pad2d, h1pk, w0_1p)

  h2pk, h2ppk = pl.pallas_call(
      _tc3, out_shape=(jax.ShapeDtypeStruct((PKH, 8 * F2), f32),
                       jax.ShapeDtypeStruct((PKH, 8 * F2), f32)))(
          dis16pk, sw16pk, agg1p.reshape(NC, PKH, 128),
          h1pk, b1t, W2b)

  agg2p, hsl2pk = _sc_agg2(ei3, pad2d, h2ppk.reshape(NP, F2),
                           h2pk, w0_2p, zeros48)

  outp = pl.pallas_call(
      _tc4, out_shape=jax.ShapeDtypeStruct((PKH, 8 * F2), f32))(
          dis16pk, agg2p.reshape(NC, PKH, 8 * F2),
          hsl2pk, b2t)
  return outp.reshape(NP, F2)[:N, :C]


# speedup vs baseline: 2.0788x; 1.0000x over previous
import functools

import jax
import jax.numpy as jnp
from jax import lax
from jax.experimental import pallas as pl
from jax.experimental.pallas import tpu as pltpu
from jax.experimental.pallas import tpu_sc as plsc

N = 10000
E = 320000
D_IN = 128
H = 16
C = 40

NP = 10240
F2 = 48

NC = 2
NS = 16
NW = NC * NS
K = 128
NR = E // K
CH = 80
CHD = 160
NPAD = NW * CH - NR
CHQ = CH // 4
RP = NP // NS
HRP = RP // 2
DEG_Q = 32

PKH = NP * H // 128
PKF = F2 // H

_mesh = plsc.VectorSubcoreMesh(core_axis_name="c", subcore_axis_name="s")
_sc_params = pltpu.CompilerParams(use_tc_tiling_on_sc=False,
                                  needs_layout_passes=False)


def _agg_pipeline(tab, idxr_v, idxc_v, buf, acc_sp, semg, sems):
  def g(j, q):
    pltpu.async_copy(tab.at[idxr_v.at[j]], buf.at[q], semg[q])

  def sct(j, p):
    pltpu.async_copy(buf.at[p], acc_sp.at[idxc_v.at[j]], sems[p], add=True)

  def wait_g(q):
    pltpu.make_async_copy(tab.at[pl.ds(0, K)], buf.at[q], semg[q]).wait()

  def wait_s(p):
    pltpu.make_async_copy(buf.at[p], acc_sp.at[pl.ds(0, K)], sems[p]).wait()

  g(0, 0)
  g(1, 1)
  wait_g(0); sct(0, 0); g(2, 2)
  wait_g(1); sct(1, 1); g(3, 3)
  wait_g(2); sct(2, 2); wait_s(0); g(4, 0)
  wait_g(3); sct(3, 3); wait_s(1); g(5, 1)

  def body(t, _):
    for p in range(4):
      j = 4 * t + p
      q = (p + 2) % 4
      wait_g(p)
      sct(j, p)
      wait_s(q)
      g(j + 2, q)
    return 0

  lax.fori_loop(1, CHQ - 1, body, 0)

  for p in range(4):
    j = 4 * (CHQ - 1) + p
    q = (p + 2) % 4
    wait_g(p)
    sct(j, p)
    wait_s(q)
    if j + 2 < CH:
      g(j + 2, q)
  wait_s(2)
  wait_s(3)


def _load_agg_idx(dst, ei3, d, pad, w, sem):
  @pl.when(w < NW - 1)
  def _():
    pltpu.async_copy(ei3.at[d, pl.ds(w * CH, CH)], dst, sem)

  @pl.when(w == NW - 1)
  def _():
    nreal = NR - (NW - 1) * CH
    pltpu.async_copy(ei3.at[d, pl.ds((NW - 1) * CH, nreal)],
                     dst.at[pl.ds(0, nreal)], sem)
    pltpu.async_copy(pad.at[pl.ds(0, NPAD)],
                     dst.at[pl.ds(nreal, NPAD)], sem)


def _sigmoid16(src_v, dst_v):
  def body(i, _):
    v = src_v[pl.ds(i * 16, 16)]
    dst_v[pl.ds(i * 16, 16)] = 1.0 / (1.0 + jnp.exp(-v))
    return 0

  lax.fori_loop(0, RP // 16, body, 0)


def _sc_fused_layer1(ei3, pad2d, h1, w01):
  @functools.partial(
      pl.kernel,
      out_type=(jax.ShapeDtypeStruct((NC, NP, H), jnp.float32),
                jax.ShapeDtypeStruct((PKH, 128), jnp.float32),
                jax.ShapeDtypeStruct((PKH, 128), jnp.float32)),
      mesh=_mesh,
      compiler_params=_sc_params,
      scratch_types=[
          pltpu.VMEM((CHD, K), jnp.int32),
          pltpu.VMEM((CH, K), jnp.int32),
          pltpu.VMEM((CH, K), jnp.int32),
          pltpu.VMEM((4, K, H), jnp.float32),
          pltpu.VMEM((RP, H), jnp.float32),
          pltpu.VMEM((RP // 8, 128), jnp.float32),
          pltpu.VMEM((RP, H), jnp.float32),
          pltpu.VMEM((RP // 8, 128), jnp.float32),
          pltpu.VMEM((RP // 8, 128), jnp.float32),
          pltpu.VMEM((RP,), jnp.float32),
          pltpu.VMEM((RP,), jnp.float32),
          pltpu.VMEM((RP,), jnp.float32),
          pltpu.VMEM((K, H), jnp.float32),
          pltpu.VMEM_SHARED((NP, H), jnp.float32),
          pltpu.VMEM_SHARED((NP, H), jnp.float32),
      ] + [pltpu.SemaphoreType.DMA] * 14,
  )
  def k(ei3_hbm, pad_hbm, h1_hbm, w0_hbm,
        agg_out, dis16_out, sw16_out,
        idxd, idxr_v, idxc_v, buf, degst, h1v, h1pv, disrow, swrow,
        disf, w0v, swf, onesv, acc_sp, h1p_sp, *sems14):
    si0, si1, si2, si3, si4, semd = sems14[0:6]
    semg = sems14[6:10]
    sems = sems14[10:14]
    c = lax.axis_index("c")
    s = lax.axis_index("s")
    w = s * NC + c

    @pl.when(s < NS - 1)
    def _():
      pltpu.async_copy(ei3_hbm.at[1, pl.ds(s * CHD, CHD)], idxd, si0)

    @pl.when(s == NS - 1)
    def _():
      nreal = NR - (NS - 1) * CHD
      pltpu.async_copy(ei3_hbm.at[1, pl.ds((NS - 1) * CHD, nreal)],
                       idxd.at[pl.ds(0, nreal)], si0)
      pltpu.async_copy(pad_hbm.at[pl.ds(0, NPAD)],
                       idxd.at[pl.ds(nreal, NPAD)], si0)

    _load_agg_idx(idxr_v, ei3_hbm, 0, pad_hbm, w, si1)
    _load_agg_idx(idxc_v, ei3_hbm, 1, pad_hbm, w, si2)
    ld3 = pltpu.async_copy(h1_hbm.at[pl.ds(s * (RP // 8), RP // 8)], h1v, si3)
    ld4 = pltpu.async_copy(w0_hbm.at[pl.ds(s * RP, RP)], w0v, si4)

    def fill_ones(i, _):
      onesv[i, :] = jnp.ones((H,), jnp.float32)
      return 0

    lax.fori_loop(0, K, fill_ones, 0)

    def fill_zero(i, _):
      degst[i, :] = jnp.zeros((H,), jnp.float32)
      return 0

    lax.fori_loop(0, RP, fill_zero, 0)
    pltpu.sync_copy(degst, acc_sp.at[pl.ds(s * RP, RP)])
    pltpu.make_async_copy(ei3_hbm.at[1, pl.ds(0, CHD)], idxd, si0).wait()
    plsc.subcore_barrier()

    def wait_one_deg():
      pltpu.make_async_copy(onesv, acc_sp.at[pl.ds(0, K)], semd).wait()

    def dbody(j, _):
      pltpu.async_copy(onesv, acc_sp.at[idxd.at[j]], semd, add=True)

      @pl.when(j >= DEG_Q)
      def _():
        wait_one_deg()

      return 0

    lax.fori_loop(0, CHD, dbody, 0)

    def drain(j, _):
      wait_one_deg()
      return 0

    lax.fori_loop(0, DEG_Q, drain, 0)
    plsc.subcore_barrier()

    pltpu.sync_copy(acc_sp.at[pl.ds(s * RP, RP)], degst)
    iota = lax.iota(jnp.int32, 16)
    zi = jnp.zeros((16,), jnp.int32)

    def disbody(i, _):
      v = plsc.load_gather(degst, [i * 16 + iota, zi])
      bits = lax.bitcast_convert_type(v, jnp.int32)
      y = lax.bitcast_convert_type(jnp.int32(0x5F3759DF) - (bits >> 1),
                                   jnp.float32)
      for _ in range(4):
        y = y * (1.5 - 0.5 * v * y * y)
      y = jnp.where(v > 0, y, 0.0)
      disf[pl.ds(i * 16, 16)] = y
      return 0

    lax.fori_loop(0, RP // 16, disbody, 0)

    ld4.wait()
    _sigmoid16(w0v, swf)

    ld3.wait()

    def scale(ri, _):
      for a in range(8):
        r = 8 * ri + a
        db = plsc.load_gather(disf, [jnp.full((16,), r, jnp.int32)])
        sb = plsc.load_gather(swf, [jnp.full((16,), r, jnp.int32)])
        h1pv[r, :] = h1v[ri, pl.ds(a * H, H)] * db
        disrow[ri, pl.ds(a * H, H)] = db
        swrow[ri, pl.ds(a * H, H)] = sb
      return 0

    lax.fori_loop(0, RP // 8, scale, 0)
    pltpu.sync_copy(h1pv, h1p_sp.at[pl.ds(s * RP, RP)])
    half = s * RP + c * HRP
    phalf = s * (RP // 8) + c * (HRP // 8)
    pltpu.sync_copy(disrow.at[pl.ds(c * (HRP // 8), HRP // 8)],
                    dis16_out.at[pl.ds(phalf, HRP // 8)])
    pltpu.sync_copy(swrow.at[pl.ds(c * (HRP // 8), HRP // 8)],
                    sw16_out.at[pl.ds(phalf, HRP // 8)])
    pltpu.make_async_copy(ei3_hbm.at[0, pl.ds(0, CH)], idxr_v, si1).wait()
    pltpu.make_async_copy(ei3_hbm.at[1, pl.ds(0, CH)], idxc_v, si2).wait()
    plsc.subcore_barrier()
    def fill_zero2(i, _):
      degst[i, :] = jnp.zeros((H,), jnp.float32)
      return 0

    lax.fori_loop(0, RP, fill_zero2, 0)
    pltpu.sync_copy(degst, acc_sp.at[pl.ds(s * RP, RP)])
    plsc.subcore_barrier()

    _agg_pipeline(h1p_sp, idxr_v, idxc_v, buf, acc_sp, semg, sems)
    plsc.subcore_barrier()
    pltpu.sync_copy(acc_sp.at[pl.ds(s * RP, RP)],
                    agg_out.at[c, pl.ds(s * RP, RP)])

  return k(ei3, pad2d, h1, w01)


PKQ = NP * F2 // 384


def _sc_agg2(ei3, pad2d, table, h2pk, w02, zeros48):
  @functools.partial(
      pl.kernel,
      out_type=(jax.ShapeDtypeStruct((NC, NP, F2), jnp.float32),
                jax.ShapeDtypeStruct((PKQ, 384), jnp.float32)),
      mesh=_mesh,
      compiler_params=_sc_params,
      scratch_types=[
          pltpu.VMEM((CH, K), jnp.int32),
          pltpu.VMEM((CH, K), jnp.int32),
          pltpu.VMEM((4, K, F2), jnp.float32),
          pltpu.VMEM((RP, F2), jnp.float32),
          pltpu.VMEM((HRP // 8, 384), jnp.float32),
          pltpu.VMEM((RP,), jnp.float32),
          pltpu.VMEM((RP,), jnp.float32),
          pltpu.VMEM_SHARED((NP, F2), jnp.float32),
      ] + [pltpu.SemaphoreType.DMA] * 11,
  )
  def k(ei3_hbm, pad_hbm, tab_hbm, h2_hbm, w0_hbm, z_hbm,
        out_hbm, hsl_out,
        idxr_v, idxc_v, buf, zerov, h2v, w0v, swf, acc_sp, *sems11):
    si1, si2, si4 = sems11[0:3]
    semg = sems11[3:7]
    sems = sems11[7:11]
    c = lax.axis_index("c")
    s = lax.axis_index("s")
    w = s * NC + c

    _load_agg_idx(idxr_v, ei3_hbm, 0, pad_hbm, w, si1)
    _load_agg_idx(idxc_v, ei3_hbm, 1, pad_hbm, w, si2)
    ld4 = pltpu.async_copy(w0_hbm.at[pl.ds(s * RP, RP)], w0v, si4)
    phalf = s * (RP // 8) + c * (HRP // 8)
    ld5 = pltpu.async_copy(h2_hbm.at[pl.ds(phalf, HRP // 8)], h2v, si4)
    pltpu.sync_copy(z_hbm.at[pl.ds(s * RP, RP)], zerov)
    pltpu.sync_copy(zerov, acc_sp.at[pl.ds(s * RP, RP)])
    ld4.wait()
    ld5.wait()
    _sigmoid16(w0v, swf)

    def hsl(ri, _):
      for a in range(8):
        r = 8 * ri + a + c * HRP
        sb = plsc.load_gather(swf, [jnp.full((16,), r, jnp.int32)])
        for b in range(PKF):
          off = a * F2 + b * H
          h2v[ri, pl.ds(off, H)] = h2v[ri, pl.ds(off, H)] * sb
      return 0

    lax.fori_loop(0, HRP // 8, hsl, 0)
    pltpu.sync_copy(h2v, hsl_out.at[pl.ds(phalf, HRP // 8)])
    pltpu.make_async_copy(ei3_hbm.at[0, pl.ds(0, CH)], idxr_v, si1).wait()
    pltpu.make_async_copy(ei3_hbm.at[1, pl.ds(0, CH)], idxc_v, si2).wait()
    plsc.subcore_barrier()

    _agg_pipeline(tab_hbm, idxr_v, idxc_v, buf, acc_sp, semg, sems)
    plsc.subcore_barrier()
    pltpu.sync_copy(acc_sp.at[pl.ds(s * RP, RP)],
                    out_hbm.at[c, pl.ds(s * RP, RP)])

  return k(ei3, pad2d, table, h2pk, w02, zeros48)


def _tc1(xr_ref, w1b_ref, h1_ref):
  h1_ref[pl.ds(0, N // 8), :] = jnp.dot(xr_ref[...], w1b_ref[...],
                                        preferred_element_type=jnp.float32)
  h1_ref[pl.ds(N // 8, (NP - N) // 8), :] = jnp.zeros(((NP - N) // 8, 128),
                                                      jnp.float32)


def _dis48_from_packed(dis16):
  return jnp.concatenate(
      [jnp.broadcast_to(dis16[:, 16 * a:16 * a + 1], (PKH, F2))
       for a in range(8)], axis=1)


def _tc3(dis16_ref, sw16_ref, aggp_ref, h1_ref, b1_ref, w2b_ref,
         h2_ref, h2p_ref):
  agg = aggp_ref[0] + aggp_ref[1]
  dis16 = dis16_ref[...]
  z = dis16 * agg + sw16_ref[...] * h1_ref[...] + b1_ref[...]
  z = jnp.maximum(z, 0.0)
  h2 = jnp.dot(z, w2b_ref[...], preferred_element_type=jnp.float32)
  h2_ref[...] = h2
  h2p_ref[...] = h2 * _dis48_from_packed(dis16)


def _tc4(dis16_ref, aggp_ref, hsl2_ref, b2_ref, o_ref):
  agg = aggp_ref[0] + aggp_ref[1]
  o_ref[...] = (_dis48_from_packed(dis16_ref[...]) * agg + hsl2_ref[...]
                + b2_ref[...])


def kernel(x, edge_index, w0_1, W1, b1, w0_2, W2, b2):
  f32 = jnp.float32
  ei3 = edge_index.astype(jnp.int32).reshape(2, NR, K)
  pad2d = (jnp.arange(NPAD * K, dtype=jnp.int32) % (NP - N) + N).reshape(
      NPAD, K)

  w0_1p = jnp.pad(w0_1, (0, NP - N))
  w0_2p = jnp.pad(w0_2, (0, NP - N))
  eye8 = jnp.eye(8, dtype=f32)
  W1b = (eye8[:, None, :, None] * W1[None, :, None, :]).reshape(8 * D_IN,
                                                                8 * H)
  W2p = jnp.pad(W2, ((0, 0), (0, F2 - C)))
  W2b = (eye8[:, None, :, None] * W2p[None, :, None, :]).reshape(8 * H,
                                                                 8 * F2)
  b1t = jnp.tile(b1, 8).reshape(1, 128)
  b2t = jnp.tile(jnp.pad(b2, (0, F2 - C)), 8).reshape(1, 8 * F2)
  zeros48 = jnp.zeros((NP, F2), f32)

  xr = x.reshape(N // 8, 8 * D_IN)
  h1pk = pl.pallas_call(
      _tc1, out_shape=jax.ShapeDtypeStruct((PKH, 128), f32))(xr, W1b)

  agg1p, dis16pk, sw16pk = _sc_fused_layer1(
      ei3, pad2d, h1pk, w0_1p)

  h2pk, h2ppk = pl.pallas_call(
      _tc3, out_shape=(jax.ShapeDtypeStruct((PKH, 8 * F2), f32),
                       jax.ShapeDtypeStruct((PKH, 8 * F2), f32)))(
          dis16pk, sw16pk, agg1p.reshape(NC, PKH, 128),
          h1pk, b1t, W2b)

  agg2p, hsl2pk = _sc_agg2(ei3, pad2d, h2ppk.reshape(NP, F2),
                           h2pk, w0_2p, zeros48)

  outp = pl.pallas_call(
      _tc4, out_shape=jax.ShapeDtypeStruct((PKH, 8 * F2), f32))(
          dis16pk, agg2p.reshape(NC, PKH, 8 * F2),
          hsl2pk, b2t)
  return outp.reshape(NP, F2)[:N, :C]


# speedup vs baseline: 2.2977x; 1.1053x over previous
import functools

import jax
import jax.numpy as jnp
from jax import lax
from jax.experimental import pallas as pl
from jax.experimental.pallas import tpu as pltpu
from jax.experimental.pallas import tpu_sc as plsc

N = 10000
E = 320000
D_IN = 128
H = 16
C = 40

NP = 10240
F2 = 48

NC = 2
NS = 16
NW = NC * NS
K = 128
NR = E // K
CH = 80
CHD = 160
NPAD = NW * CH - NR
CHQ = CH // 4
RP = NP // NS
HRP = RP // 2
DEG_Q = 32

PKH = NP * H // 128
PKF = F2 // H

_mesh = plsc.VectorSubcoreMesh(core_axis_name="c", subcore_axis_name="s")
_sc_params = pltpu.CompilerParams(use_tc_tiling_on_sc=False,
                                  needs_layout_passes=False)


def _agg_pipeline8(tab, idxr_v, idxc_v, buf, acc_sp, semg, sems):
  def g(j, q):
    pltpu.async_copy(tab.at[idxr_v.at[j]], buf.at[q], semg[q])

  def sct(j, p):
    pltpu.async_copy(buf.at[p], acc_sp.at[idxc_v.at[j]], sems[p], add=True)

  def wait_g(q):
    pltpu.make_async_copy(tab.at[pl.ds(0, K)], buf.at[q], semg[q]).wait()

  def wait_s(p):
    pltpu.make_async_copy(buf.at[p], acc_sp.at[pl.ds(0, K)], sems[p]).wait()

  for q in range(6):
    g(q, q)

  def step(j, p):
    q = (p + 6) % 8
    wait_g(p)
    sct(j, p)
    wait_s(q)
    g(j + 6, q)

  wait_g(0); sct(0, 0); g(6, 6)
  wait_g(1); sct(1, 1); g(7, 7)
  for p in range(2, 8):
    step(p, p)

  def body(t, _):
    for p in range(8):
      step(8 * t + p, p)
    return 0

  lax.fori_loop(1, CH // 8 - 1, body, 0)

  for p in range(8):
    j = CH - 8 + p
    q = (p + 6) % 8
    wait_g(p)
    sct(j, p)
    wait_s(q)
    if j + 6 < CH:
      g(j + 6, q)
  wait_s(6)
  wait_s(7)


def _agg_pipeline4(tab, idxr_v, idxc_v, buf, acc_sp, semg, sems):
  def g(j, q):
    pltpu.async_copy(tab.at[idxr_v.at[j]], buf.at[q], semg[q])

  def sct(j, p):
    pltpu.async_copy(buf.at[p], acc_sp.at[idxc_v.at[j]], sems[p], add=True)

  def wait_g(q):
    pltpu.make_async_copy(tab.at[pl.ds(0, K)], buf.at[q], semg[q]).wait()

  def wait_s(p):
    pltpu.make_async_copy(buf.at[p], acc_sp.at[pl.ds(0, K)], sems[p]).wait()

  g(0, 0)
  g(1, 1)
  wait_g(0); sct(0, 0); g(2, 2)
  wait_g(1); sct(1, 1); g(3, 3)
  wait_g(2); sct(2, 2); wait_s(0); g(4, 0)
  wait_g(3); sct(3, 3); wait_s(1); g(5, 1)

  def body(t, _):
    for p in range(4):
      j = 4 * t + p
      q = (p + 2) % 4
      wait_g(p)
      sct(j, p)
      wait_s(q)
      g(j + 2, q)
    return 0

  lax.fori_loop(1, CHQ - 1, body, 0)

  for p in range(4):
    j = 4 * (CHQ - 1) + p
    q = (p + 2) % 4
    wait_g(p)
    sct(j, p)
    wait_s(q)
    if j + 2 < CH:
      g(j + 2, q)
  wait_s(2)
  wait_s(3)


def _load_agg_idx(dst, ei3, d, pad, w, sem):
  @pl.when(w < NW - 1)
  def _():
    pltpu.async_copy(ei3.at[d, pl.ds(w * CH, CH)], dst, sem)

  @pl.when(w == NW - 1)
  def _():
    nreal = NR - (NW - 1) * CH
    pltpu.async_copy(ei3.at[d, pl.ds((NW - 1) * CH, nreal)],
                     dst.at[pl.ds(0, nreal)], sem)
    pltpu.async_copy(pad.at[pl.ds(0, NPAD)],
                     dst.at[pl.ds(nreal, NPAD)], sem)


def _sigmoid16(src_v, dst_v):
  def body(i, _):
    v = src_v[pl.ds(i * 16, 16)]
    dst_v[pl.ds(i * 16, 16)] = 1.0 / (1.0 + jnp.exp(-v))
    return 0

  lax.fori_loop(0, RP // 16, body, 0)


def _sc_fused_layer1(ei3, pad2d, h1, w01):
  @functools.partial(
      pl.kernel,
      out_type=(jax.ShapeDtypeStruct((NC, NP, H), jnp.float32),
                jax.ShapeDtypeStruct((PKH, 128), jnp.float32),
                jax.ShapeDtypeStruct((PKH, 128), jnp.float32)),
      mesh=_mesh,
      compiler_params=_sc_params,
      scratch_types=[
          pltpu.VMEM((CHD, K), jnp.int32),
          pltpu.VMEM((CH, K), jnp.int32),
          pltpu.VMEM((CH, K), jnp.int32),
          pltpu.VMEM((4, K, H), jnp.float32),
          pltpu.VMEM((RP, H), jnp.float32),
          pltpu.VMEM((RP // 8, 128), jnp.float32),
          pltpu.VMEM((RP, H), jnp.float32),
          pltpu.VMEM((RP // 8, 128), jnp.float32),
          pltpu.VMEM((RP // 8, 128), jnp.float32),
          pltpu.VMEM((RP,), jnp.float32),
          pltpu.VMEM((RP,), jnp.float32),
          pltpu.VMEM((RP,), jnp.float32),
          pltpu.VMEM((K, H), jnp.float32),
          pltpu.VMEM_SHARED((NP, H), jnp.float32),
          pltpu.VMEM_SHARED((NP, H), jnp.float32),
      ] + [pltpu.SemaphoreType.DMA] * 14,
  )
  def k(ei3_hbm, pad_hbm, h1_hbm, w0_hbm,
        agg_out, dis16_out, sw16_out,
        idxd, idxr_v, idxc_v, buf, degst, h1v, h1pv, disrow, swrow,
        disf, w0v, swf, onesv, acc_sp, h1p_sp, *sems14):
    si0, si1, si2, si3, si4, semd = sems14[0:6]
    semg = sems14[6:10]
    sems = sems14[10:14]
    c = lax.axis_index("c")
    s = lax.axis_index("s")
    w = s * NC + c

    @pl.when(s < NS - 1)
    def _():
      pltpu.async_copy(ei3_hbm.at[1, pl.ds(s * CHD, CHD)], idxd, si0)

    @pl.when(s == NS - 1)
    def _():
      nreal = NR - (NS - 1) * CHD
      pltpu.async_copy(ei3_hbm.at[1, pl.ds((NS - 1) * CHD, nreal)],
                       idxd.at[pl.ds(0, nreal)], si0)
      pltpu.async_copy(pad_hbm.at[pl.ds(0, NPAD)],
                       idxd.at[pl.ds(nreal, NPAD)], si0)

    _load_agg_idx(idxr_v, ei3_hbm, 0, pad_hbm, w, si1)
    _load_agg_idx(idxc_v, ei3_hbm, 1, pad_hbm, w, si2)
    ld3 = pltpu.async_copy(h1_hbm.at[pl.ds(s * (RP // 8), RP // 8)], h1v, si3)
    ld4 = pltpu.async_copy(w0_hbm.at[pl.ds(s * RP, RP)], w0v, si4)

    def fill_ones(i, _):
      onesv[i, :] = jnp.ones((H,), jnp.float32)
      return 0

    lax.fori_loop(0, K, fill_ones, 0)

    def fill_zero(i, _):
      degst[i, :] = jnp.zeros((H,), jnp.float32)
      return 0

    lax.fori_loop(0, RP, fill_zero, 0)
    pltpu.sync_copy(degst, acc_sp.at[pl.ds(s * RP, RP)])
    pltpu.make_async_copy(ei3_hbm.at[1, pl.ds(0, CHD)], idxd, si0).wait()
    plsc.subcore_barrier()

    def wait_one_deg():
      pltpu.make_async_copy(onesv, acc_sp.at[pl.ds(0, K)], semd).wait()

    def dbody(j, _):
      pltpu.async_copy(onesv, acc_sp.at[idxd.at[j]], semd, add=True)

      @pl.when(j >= DEG_Q)
      def _():
        wait_one_deg()

      return 0

    lax.fori_loop(0, CHD, dbody, 0)

    def drain(j, _):
      wait_one_deg()
      return 0

    lax.fori_loop(0, DEG_Q, drain, 0)
    plsc.subcore_barrier()

    pltpu.sync_copy(acc_sp.at[pl.ds(s * RP, RP)], degst)
    iota = lax.iota(jnp.int32, 16)
    zi = jnp.zeros((16,), jnp.int32)

    def disbody(i, _):
      v = plsc.load_gather(degst, [i * 16 + iota, zi])
      bits = lax.bitcast_convert_type(v, jnp.int32)
      y = lax.bitcast_convert_type(jnp.int32(0x5F3759DF) - (bits >> 1),
                                   jnp.float32)
      for _ in range(4):
        y = y * (1.5 - 0.5 * v * y * y)
      y = jnp.where(v > 0, y, 0.0)
      disf[pl.ds(i * 16, 16)] = y
      return 0

    lax.fori_loop(0, RP // 16, disbody, 0)

    ld4.wait()
    _sigmoid16(w0v, swf)

    ld3.wait()

    def scale(ri, _):
      for a in range(8):
        r = 8 * ri + a
        db = plsc.load_gather(disf, [jnp.full((16,), r, jnp.int32)])
        sb = plsc.load_gather(swf, [jnp.full((16,), r, jnp.int32)])
        h1pv[r, :] = h1v[ri, pl.ds(a * H, H)] * db
        disrow[ri, pl.ds(a * H, H)] = db
        swrow[ri, pl.ds(a * H, H)] = sb
      return 0

    lax.fori_loop(0, RP // 8, scale, 0)
    pltpu.sync_copy(h1pv, h1p_sp.at[pl.ds(s * RP, RP)])
    half = s * RP + c * HRP
    phalf = s * (RP // 8) + c * (HRP // 8)
    pltpu.sync_copy(disrow.at[pl.ds(c * (HRP // 8), HRP // 8)],
                    dis16_out.at[pl.ds(phalf, HRP // 8)])
    pltpu.sync_copy(swrow.at[pl.ds(c * (HRP // 8), HRP // 8)],
                    sw16_out.at[pl.ds(phalf, HRP // 8)])
    pltpu.make_async_copy(ei3_hbm.at[0, pl.ds(0, CH)], idxr_v, si1).wait()
    pltpu.make_async_copy(ei3_hbm.at[1, pl.ds(0, CH)], idxc_v, si2).wait()
    plsc.subcore_barrier()
    def fill_zero2(i, _):
      degst[i, :] = jnp.zeros((H,), jnp.float32)
      return 0

    lax.fori_loop(0, RP, fill_zero2, 0)
    pltpu.sync_copy(degst, acc_sp.at[pl.ds(s * RP, RP)])
    plsc.subcore_barrier()

    _agg_pipeline4(h1p_sp, idxr_v, idxc_v, buf, acc_sp, semg, sems)
    plsc.subcore_barrier()
    pltpu.sync_copy(acc_sp.at[pl.ds(s * RP, RP)],
                    agg_out.at[c, pl.ds(s * RP, RP)])

  return k(ei3, pad2d, h1, w01)


PKQ = NP * F2 // 384


def _sc_agg2(ei3, pad2d, table, h2pk, w02):
  @functools.partial(
      pl.kernel,
      out_type=(jax.ShapeDtypeStruct((NC, NP, F2), jnp.float32),
                jax.ShapeDtypeStruct((PKQ, 384), jnp.float32)),
      mesh=_mesh,
      compiler_params=_sc_params,
      scratch_types=[
          pltpu.VMEM((CH, K), jnp.int32),
          pltpu.VMEM((CH, K), jnp.int32),
          pltpu.VMEM((8, K, F2), jnp.float32),
          pltpu.VMEM((40, F2), jnp.float32),
          pltpu.VMEM((HRP // 8, 384), jnp.float32),
          pltpu.VMEM((RP,), jnp.float32),
          pltpu.VMEM((RP,), jnp.float32),
          pltpu.VMEM_SHARED((NP, F2), jnp.float32),
      ] + [pltpu.SemaphoreType.DMA] * 19,
  )
  def k(ei3_hbm, pad_hbm, tab_hbm, h2_hbm, w0_hbm,
        out_hbm, hsl_out,
        idxr_v, idxc_v, buf, zerov, h2v, w0v, swf, acc_sp, *sems11):
    si1, si2, si4 = sems11[0:3]
    semg = sems11[3:11]
    sems = sems11[11:19]
    c = lax.axis_index("c")
    s = lax.axis_index("s")
    w = s * NC + c

    _load_agg_idx(idxr_v, ei3_hbm, 0, pad_hbm, w, si1)
    _load_agg_idx(idxc_v, ei3_hbm, 1, pad_hbm, w, si2)
    ld4 = pltpu.async_copy(w0_hbm.at[pl.ds(s * RP, RP)], w0v, si4)
    phalf = s * (RP // 8) + c * (HRP // 8)
    ld5 = pltpu.async_copy(h2_hbm.at[pl.ds(phalf, HRP // 8)], h2v, si4)

    def fill_zero(i, _):
      for b in range(PKF):
        zerov[i, pl.ds(b * H, H)] = jnp.zeros((H,), jnp.float32)
      return 0

    lax.fori_loop(0, 40, fill_zero, 0)
    for t in range(RP // 40):
      pltpu.sync_copy(zerov, acc_sp.at[pl.ds(s * RP + t * 40, 40)])
    ld4.wait()
    ld5.wait()
    _sigmoid16(w0v, swf)

    def hsl(ri, _):
      for a in range(8):
        r = 8 * ri + a + c * HRP
        sb = plsc.load_gather(swf, [jnp.full((16,), r, jnp.int32)])
        for b in range(PKF):
          off = a * F2 + b * H
          h2v[ri, pl.ds(off, H)] = h2v[ri, pl.ds(off, H)] * sb
      return 0

    lax.fori_loop(0, HRP // 8, hsl, 0)
    pltpu.sync_copy(h2v, hsl_out.at[pl.ds(phalf, HRP // 8)])
    pltpu.make_async_copy(ei3_hbm.at[0, pl.ds(0, CH)], idxr_v, si1).wait()
    pltpu.make_async_copy(ei3_hbm.at[1, pl.ds(0, CH)], idxc_v, si2).wait()
    plsc.subcore_barrier()

    _agg_pipeline8(tab_hbm, idxr_v, idxc_v, buf, acc_sp, semg, sems)
    plsc.subcore_barrier()
    pltpu.sync_copy(acc_sp.at[pl.ds(s * RP, RP)],
                    out_hbm.at[c, pl.ds(s * RP, RP)])

  return k(ei3, pad2d, table, h2pk, w02)


def _tc1(xr_ref, w1b_ref, h1_ref):
  h1_ref[pl.ds(0, N // 8), :] = jnp.dot(xr_ref[...], w1b_ref[...],
                                        preferred_element_type=jnp.float32)
  h1_ref[pl.ds(N // 8, (NP - N) // 8), :] = jnp.zeros(((NP - N) // 8, 128),
                                                      jnp.float32)


def _dis48_from_packed(dis16):
  return jnp.concatenate(
      [jnp.broadcast_to(dis16[:, 16 * a:16 * a + 1], (PKH, F2))
       for a in range(8)], axis=1)


def _tc3(dis16_ref, sw16_ref, aggp_ref, h1_ref, b1_ref, w2b_ref,
         h2_ref, h2p_ref):
  agg = aggp_ref[0] + aggp_ref[1]
  dis16 = dis16_ref[...]
  z = dis16 * agg + sw16_ref[...] * h1_ref[...] + b1_ref[...]
  z = jnp.maximum(z, 0.0)
  h2 = jnp.dot(z, w2b_ref[...], preferred_element_type=jnp.float32)
  h2_ref[...] = h2
  h2p_ref[...] = h2 * _dis48_from_packed(dis16)


def _tc4(dis16_ref, aggp_ref, hsl2_ref, b2_ref, o_ref):
  agg = aggp_ref[0] + aggp_ref[1]
  o_ref[...] = (_dis48_from_packed(dis16_ref[...]) * agg + hsl2_ref[...]
                + b2_ref[...])


def kernel(x, edge_index, w0_1, W1, b1, w0_2, W2, b2):
  f32 = jnp.float32
  ei3 = edge_index.astype(jnp.int32).reshape(2, NR, K)
  pad2d = (jnp.arange(NPAD * K, dtype=jnp.int32) % (NP - N) + N).reshape(
      NPAD, K)

  w0_1p = jnp.pad(w0_1, (0, NP - N))
  w0_2p = jnp.pad(w0_2, (0, NP - N))
  eye8 = jnp.eye(8, dtype=f32)
  W1b = (eye8[:, None, :, None] * W1[None, :, None, :]).reshape(8 * D_IN,
                                                                8 * H)
  W2p = jnp.pad(W2, ((0, 0), (0, F2 - C)))
  W2b = (eye8[:, None, :, None] * W2p[None, :, None, :]).reshape(8 * H,
                                                                 8 * F2)
  b1t = jnp.tile(b1, 8).reshape(1, 128)
  b2t = jnp.tile(jnp.pad(b2, (0, F2 - C)), 8).reshape(1, 8 * F2)

  xr = x.reshape(N // 8, 8 * D_IN)
  h1pk = pl.pallas_call(
      _tc1, out_shape=jax.ShapeDtypeStruct((PKH, 128), f32))(xr, W1b)

  agg1p, dis16pk, sw16pk = _sc_fused_layer1(
      ei3, pad2d, h1pk, w0_1p)

  h2pk, h2ppk = pl.pallas_call(
      _tc3, out_shape=(jax.ShapeDtypeStruct((PKH, 8 * F2), f32),
                       jax.ShapeDtypeStruct((PKH, 8 * F2), f32)))(
          dis16pk, sw16pk, agg1p.reshape(NC, PKH, 128),
          h1pk, b1t, W2b)

  agg2p, hsl2pk = _sc_agg2(ei3, pad2d, h2ppk.reshape(NP, F2),
                           h2pk, w0_2p)

  outp = pl.pallas_call(
      _tc4, out_shape=jax.ShapeDtypeStruct((PKH, 8 * F2), f32))(
          dis16pk, agg2p.reshape(NC, PKH, 8 * F2),
          hsl2pk, b2t)
  return outp.reshape(NP, F2)[:N, :C]
